# Initial kernel scaffold; baseline (speedup 1.0000x reference)
import jax
import jax.numpy as jnp
from jax.experimental import pallas as pl


def kernel(vision_features, vision_timestamps, proprio_features, proprio_timestamps, lang_embeddings, target_timestamps):
    raise NotImplementedError("write your pallas kernel here")


# speedup vs baseline: 20.0405x; 20.0405x over previous
import jax
import jax.numpy as jnp
from jax.experimental import pallas as pl
from jax.experimental.pallas import tpu as pltpu


def _interp_weights(times, tgt_col, t_src):
    T = tgt_col.shape[0]
    T_src = times.shape[1]
    cnt = jnp.sum((times < tgt_col).astype(jnp.int32), axis=1, keepdims=True)
    idx = jnp.minimum(cnt, t_src - 2)
    cols = jax.lax.broadcasted_iota(jnp.int32, (T, T_src), 1)
    onehot_l = (cols == idx).astype(jnp.float32)
    onehot_r = (cols == idx + 1).astype(jnp.float32)
    t_left = jnp.sum(onehot_l * times, axis=1, keepdims=True)
    t_right = jnp.sum(onehot_r * times, axis=1, keepdims=True)
    w = jnp.clip((tgt_col - t_left) / (t_right - t_left + 1e-8), 0.0, 1.0)
    return onehot_l * (1.0 - w) + onehot_r * w


def _fusion_kernel(vis_f_ref, vis_t_ref, prop_f_ref, prop_t_ref,
                   lang_ref, tgt_ref, out_ref):
    tgt = tgt_ref[0]
    tgt_col = tgt.reshape(tgt.shape[1], 1)
    w_vis = _interp_weights(vis_t_ref[0], tgt_col, vis_f_ref.shape[1])
    w_prop = _interp_weights(prop_t_ref[0], tgt_col, prop_f_ref.shape[1])
    vis = jnp.dot(w_vis, vis_f_ref[0], preferred_element_type=jnp.float32)
    prop = jnp.dot(w_prop, prop_f_ref[0], preferred_element_type=jnp.float32)
    lang_avg = jnp.mean(lang_ref[0], axis=0, keepdims=True)
    lang_b = jnp.broadcast_to(lang_avg, (tgt.shape[1], lang_avg.shape[1]))
    out_ref[0] = jnp.concatenate([vis, prop, lang_b], axis=-1)


def kernel(vision_features, vision_timestamps, proprio_features,
           proprio_timestamps, lang_embeddings, target_timestamps):
    B, T_vis, D_vis = vision_features.shape
    _, T_prop, D_prop = proprio_features.shape
    _, L, D_lang = lang_embeddings.shape
    T = target_timestamps.shape[1]
    D_out = D_vis + D_prop + D_lang

    vis_t3 = vision_timestamps.reshape(B, 1, T_vis)
    prop_t3 = proprio_timestamps.reshape(B, 1, T_prop)
    tgt3 = target_timestamps.reshape(B, 1, T)

    out = pl.pallas_call(
        _fusion_kernel,
        grid=(B,),
        in_specs=[
            pl.BlockSpec((1, T_vis, D_vis), lambda b: (b, 0, 0)),
            pl.BlockSpec((1, 1, T_vis), lambda b: (b, 0, 0)),
            pl.BlockSpec((1, T_prop, D_prop), lambda b: (b, 0, 0)),
            pl.BlockSpec((1, 1, T_prop), lambda b: (b, 0, 0)),
            pl.BlockSpec((1, L, D_lang), lambda b: (b, 0, 0)),
            pl.BlockSpec((1, 1, T), lambda b: (b, 0, 0)),
        ],
        out_specs=pl.BlockSpec((1, T, D_out), lambda b: (b, 0, 0)),
        out_shape=jax.ShapeDtypeStruct((B, T, D_out), jnp.float32),
        compiler_params=pltpu.CompilerParams(
            dimension_semantics=("arbitrary",),
        ),
    )(vision_features, vis_t3, proprio_features, prop_t3,
      lang_embeddings, tgt3)
    return out
